# baseline (speedup 1.0000x reference)
import jax
import jax.numpy as jnp
from jax.experimental import pallas as pl
from jax.experimental.pallas import tpu as pltpu

DIM_K = 2048
NG = 16
GS = 4096
CAP = 320
QW = NG * CAP
TB1 = 512
NB1 = GS // TB1
TB2 = 128
NB2 = GS // TB2
EPS_ = 1e-9
NEG_BIG = -3.4e38


def _p1_body(x_ref, w_ref, sm_out, cnt_out, sum_out, acc_ref):
    i = pl.program_id(0)

    @pl.when(i == 0)
    def _init():
        acc_ref[...] = jnp.zeros_like(acc_ref)

    raw = jnp.dot(x_ref[...], w_ref[...],
                  preferred_element_type=jnp.float32)
    m = jnp.max(raw, axis=1, keepdims=True)
    e = jnp.exp(raw - m)
    sm = e / jnp.sum(e, axis=1, keepdims=True)
    sm_out[...] = sm
    g1 = jnp.max(sm, axis=1, keepdims=True)
    iota = jax.lax.broadcasted_iota(jnp.int32, (TB1, NG), 1)
    i1 = jnp.min(jnp.where(sm == g1, iota, NG), axis=1, keepdims=True)
    mask1 = (iota == i1).astype(jnp.float32)
    acc_ref[0:1, :] += jnp.sum(mask1, axis=0, keepdims=True)
    acc_ref[1:2, :] += jnp.sum(sm, axis=0, keepdims=True)

    @pl.when(i == NB1 - 1)
    def _fin():
        cnt_out[...] = acc_ref[0:1, :]
        sum_out[...] = acc_ref[1:2, :]


def _top2(sm):
    iota = jax.lax.broadcasted_iota(jnp.int32, sm.shape, 1)
    g1 = jnp.max(sm, axis=1, keepdims=True)
    i1 = jnp.min(jnp.where(sm == g1, iota, NG), axis=1, keepdims=True)
    masked = jnp.where(iota == i1, NEG_BIG, sm)
    g2 = jnp.max(masked, axis=1, keepdims=True)
    i2 = jnp.min(jnp.where(masked == g2, iota, NG), axis=1, keepdims=True)
    return g1, i1, g2, i2, iota


def _p2_body(sm_ref, cnt_ref, sum_ref, disp_ref, comb_ref,
             loss_ref, c1_ref, c2_ref, acc_ref):
    j = pl.program_id(0)

    @pl.when(j == 0)
    def _init():
        acc_ref[...] = jnp.zeros_like(acc_ref)

    sm = sm_ref[...]
    g1, i1, g2, i2, iota = _top2(sm)
    mask1 = (iota == i1).astype(jnp.float32)
    mask2 = (iota == i2).astype(jnp.float32)

    r = jax.lax.broadcasted_iota(jnp.int32, (TB2, TB2), 0)
    c = jax.lax.broadcasted_iota(jnp.int32, (TB2, TB2), 1)
    tril = (r > c).astype(jnp.float32)
    prev1 = jnp.dot(tril, mask1, preferred_element_type=jnp.float32)
    prev2 = jnp.dot(tril, mask2, preferred_element_type=jnp.float32)

    c1_run = acc_ref[0:1, :]
    c2_run = acc_ref[1:2, :]
    m1cnt = jnp.minimum(cnt_ref[...], float(CAP))

    pos1 = jnp.sum((c1_run + prev1) * mask1, axis=1, keepdims=True)
    keep1 = (pos1 < float(CAP)).astype(jnp.float32)
    pos2 = jnp.sum((c2_run + prev2 + m1cnt) * mask2, axis=1, keepdims=True)
    keep2 = (pos2 < float(CAP)).astype(jnp.float32)

    acc_ref[0:1, :] += jnp.sum(mask1, axis=0, keepdims=True)
    acc_ref[1:2, :] += jnp.sum(mask2, axis=0, keepdims=True)
    acc_ref[2:3, :] += jnp.sum(mask2 * keep2, axis=0, keepdims=True)

    denom = g1 + g2 + EPS_
    g1k = (g1 / denom) * keep1
    g2k = (g2 / denom) * keep2

    piota = jax.lax.broadcasted_iota(jnp.int32, (TB2, CAP), 1)
    ohp1 = (piota == pos1.astype(jnp.int32)).astype(jnp.float32)
    ohp2 = (piota == pos2.astype(jnp.int32)).astype(jnp.float32)
    a1 = g1k * mask1
    a2 = g2k * mask2
    comb = (a1[:, :, None] * ohp1[:, None, :]
            + a2[:, :, None] * ohp2[:, None, :])
    comb_ref[...] = comb
    disp_ref[...] = (comb != 0.0).astype(jnp.float32)

    @pl.when(j == NB2 - 1)
    def _fin():
        c1_ref[...] = jnp.minimum(cnt_ref[...], float(CAP))
        c2_ref[...] = acc_ref[2:3, :]
        loss_ref[...] = jnp.sum(cnt_ref[...] * sum_ref[...],
                                axis=1, keepdims=True) * (
                                    float(NG) / (float(GS) * float(GS)))


def kernel(x, w_gating):
    x2 = x.reshape(GS, DIM_K)
    sm, cnt, ssum = pl.pallas_call(
        _p1_body,
        grid=(NB1,),
        in_specs=[
            pl.BlockSpec((TB1, DIM_K), lambda i: (i, 0)),
            pl.BlockSpec((DIM_K, NG), lambda i: (0, 0)),
        ],
        out_specs=[
            pl.BlockSpec((TB1, NG), lambda i: (i, 0)),
            pl.BlockSpec((1, NG), lambda i: (0, 0)),
            pl.BlockSpec((1, NG), lambda i: (0, 0)),
        ],
        out_shape=[
            jax.ShapeDtypeStruct((GS, NG), jnp.float32),
            jax.ShapeDtypeStruct((1, NG), jnp.float32),
            jax.ShapeDtypeStruct((1, NG), jnp.float32),
        ],
        scratch_shapes=[pltpu.VMEM((2, NG), jnp.float32)],
        compiler_params=pltpu.CompilerParams(
            dimension_semantics=("arbitrary",)),
    )(x2, w_gating)

    disp, comb, loss, c1, c2 = pl.pallas_call(
        _p2_body,
        grid=(NB2,),
        in_specs=[
            pl.BlockSpec((TB2, NG), lambda j: (j, 0)),
            pl.BlockSpec((1, NG), lambda j: (0, 0)),
            pl.BlockSpec((1, NG), lambda j: (0, 0)),
        ],
        out_specs=[
            pl.BlockSpec((TB2, NG, CAP), lambda j: (j, 0, 0)),
            pl.BlockSpec((TB2, NG, CAP), lambda j: (j, 0, 0)),
            pl.BlockSpec((1, 1), lambda j: (0, 0)),
            pl.BlockSpec((1, NG), lambda j: (0, 0)),
            pl.BlockSpec((1, NG), lambda j: (0, 0)),
        ],
        out_shape=[
            jax.ShapeDtypeStruct((GS, NG, CAP), jnp.float32),
            jax.ShapeDtypeStruct((GS, NG, CAP), jnp.float32),
            jax.ShapeDtypeStruct((1, 1), jnp.float32),
            jax.ShapeDtypeStruct((1, NG), jnp.float32),
            jax.ShapeDtypeStruct((1, NG), jnp.float32),
        ],
        scratch_shapes=[pltpu.VMEM((4, NG), jnp.float32)],
        compiler_params=pltpu.CompilerParams(
            dimension_semantics=("arbitrary",)),
    )(sm, cnt, ssum)

    return (disp[None], comb[None], loss[0, 0], c1, c2)


# speedup vs baseline: 2.6855x; 2.6855x over previous
import jax
import jax.numpy as jnp
from jax.experimental import pallas as pl
from jax.experimental.pallas import tpu as pltpu

DIM_K = 2048
NG = 16
GS = 4096
CAP = 320
TB1 = 512
NB1 = GS // TB1
TB = 128
NB = GS // TB
EPS_ = 1e-9
NEG_BIG = -3.4e38


def _p1_body(x_ref, w_ref, sm_out, cnt_out, sum_out, acc_ref):
    i = pl.program_id(0)

    @pl.when(i == 0)
    def _init():
        acc_ref[...] = jnp.zeros_like(acc_ref)

    raw = jnp.dot(x_ref[...], w_ref[...],
                  preferred_element_type=jnp.float32)
    m = jnp.max(raw, axis=1, keepdims=True)
    e = jnp.exp(raw - m)
    sm = e / jnp.sum(e, axis=1, keepdims=True)
    sm_out[...] = sm
    g1 = jnp.max(sm, axis=1, keepdims=True)
    iota = jax.lax.broadcasted_iota(jnp.int32, (TB1, NG), 1)
    i1 = jnp.min(jnp.where(sm == g1, iota, NG), axis=1, keepdims=True)
    mask1 = (iota == i1).astype(jnp.float32)
    acc_ref[0:1, :] += jnp.sum(mask1, axis=0, keepdims=True)
    acc_ref[1:2, :] += jnp.sum(sm, axis=0, keepdims=True)

    @pl.when(i == NB1 - 1)
    def _fin():
        cnt_out[...] = acc_ref[0:1, :]
        sum_out[...] = acc_ref[1:2, :]


def _p2_body(sm_ref, cnt_ref, sum_ref, r8_ref, loss_ref, c1_ref, c2_ref,
             acc_ref):
    j = pl.program_id(0)

    @pl.when(j == 0)
    def _init():
        acc_ref[...] = jnp.zeros_like(acc_ref)

    smt = jnp.transpose(sm_ref[...])
    eiota = jax.lax.broadcasted_iota(jnp.int32, (NG, TB), 0)
    g1 = jnp.max(smt, axis=0, keepdims=True)
    i1 = jnp.min(jnp.where(smt == g1, eiota, NG), axis=0, keepdims=True)
    masked = jnp.where(eiota == i1, NEG_BIG, smt)
    g2 = jnp.max(masked, axis=0, keepdims=True)
    i2 = jnp.min(jnp.where(masked == g2, eiota, NG), axis=0, keepdims=True)
    mask1 = (eiota == i1).astype(jnp.float32)
    mask2 = (eiota == i2).astype(jnp.float32)

    r = jax.lax.broadcasted_iota(jnp.int32, (TB, TB), 0)
    c = jax.lax.broadcasted_iota(jnp.int32, (TB, TB), 1)
    triu = (r < c).astype(jnp.float32)
    prev1 = jnp.dot(mask1, triu, preferred_element_type=jnp.float32)
    prev2 = jnp.dot(mask2, triu, preferred_element_type=jnp.float32)

    c1_run = acc_ref[:, 0:1]
    c2_run = acc_ref[:, 1:2]
    m1cnt = jnp.minimum(jnp.transpose(cnt_ref[...]), float(CAP))

    pos1 = jnp.sum((c1_run + prev1) * mask1, axis=0, keepdims=True)
    keep1 = (pos1 < float(CAP)).astype(jnp.float32)
    pos2 = jnp.sum((c2_run + prev2 + m1cnt) * mask2, axis=0, keepdims=True)
    keep2 = (pos2 < float(CAP)).astype(jnp.float32)

    acc_ref[:, 0:1] += jnp.sum(mask1, axis=1, keepdims=True)
    acc_ref[:, 1:2] += jnp.sum(mask2, axis=1, keepdims=True)
    acc_ref[:, 2:3] += jnp.sum(mask2 * keep2, axis=1, keepdims=True)

    denom = g1 + g2 + EPS_
    g1k = (g1 / denom) * keep1
    g2k = (g2 / denom) * keep2
    d1 = (g1k != 0.0).astype(jnp.float32)
    d2 = (g2k != 0.0).astype(jnp.float32)

    r8_ref[...] = jnp.concatenate(
        [i1.astype(jnp.float32), i2.astype(jnp.float32),
         pos1, pos2, g1k, g2k, d1, d2], axis=0)

    @pl.when(j == NB - 1)
    def _fin():
        c1_ref[...] = jnp.minimum(cnt_ref[...], float(CAP))
        c2_ref[...] = jnp.transpose(acc_ref[:, 2:3])
        loss_ref[...] = jnp.sum(cnt_ref[...] * sum_ref[...],
                                axis=1, keepdims=True) * (
                                    float(NG) / (float(GS) * float(GS)))


def _p3_body(r8_ref, disp_ref, comb_ref):
    rows = r8_ref[...]
    i1 = rows[0:1, :]
    i2 = rows[1:2, :]
    pos1 = rows[2:3, :]
    pos2 = rows[3:4, :]
    g1k = rows[4:5, :]
    g2k = rows[5:6, :]
    d1 = rows[6:7, :]
    d2 = rows[7:8, :]
    piota = jax.lax.broadcasted_iota(jnp.int32, (CAP, TB), 0).astype(jnp.float32)
    for e in range(NG):
        ef = float(e)
        is1 = i1 == ef
        is2 = i2 == ef
        pos_e = jnp.where(is1, pos1, jnp.where(is2, pos2, -1.0))
        val_e = jnp.where(is1, g1k, jnp.where(is2, g2k, 0.0))
        dva_e = jnp.where(is1, d1, jnp.where(is2, d2, 0.0))
        b = piota == pos_e
        comb_ref[e] = jnp.where(b, val_e, 0.0)
        disp_ref[e] = jnp.where(b, dva_e, 0.0)


def kernel(x, w_gating):
    x2 = x.reshape(GS, DIM_K)
    sm, cnt, ssum = pl.pallas_call(
        _p1_body,
        grid=(NB1,),
        in_specs=[
            pl.BlockSpec((TB1, DIM_K), lambda i: (i, 0)),
            pl.BlockSpec((DIM_K, NG), lambda i: (0, 0)),
        ],
        out_specs=[
            pl.BlockSpec((TB1, NG), lambda i: (i, 0)),
            pl.BlockSpec((1, NG), lambda i: (0, 0)),
            pl.BlockSpec((1, NG), lambda i: (0, 0)),
        ],
        out_shape=[
            jax.ShapeDtypeStruct((GS, NG), jnp.float32),
            jax.ShapeDtypeStruct((1, NG), jnp.float32),
            jax.ShapeDtypeStruct((1, NG), jnp.float32),
        ],
        scratch_shapes=[pltpu.VMEM((2, NG), jnp.float32)],
        compiler_params=pltpu.CompilerParams(
            dimension_semantics=("arbitrary",)),
    )(x2, w_gating)

    r8, loss, c1, c2 = pl.pallas_call(
        _p2_body,
        grid=(NB,),
        in_specs=[
            pl.BlockSpec((TB, NG), lambda j: (j, 0)),
            pl.BlockSpec((1, NG), lambda j: (0, 0)),
            pl.BlockSpec((1, NG), lambda j: (0, 0)),
        ],
        out_specs=[
            pl.BlockSpec((8, TB), lambda j: (0, j)),
            pl.BlockSpec((1, 1), lambda j: (0, 0)),
            pl.BlockSpec((1, NG), lambda j: (0, 0)),
            pl.BlockSpec((1, NG), lambda j: (0, 0)),
        ],
        out_shape=[
            jax.ShapeDtypeStruct((8, GS), jnp.float32),
            jax.ShapeDtypeStruct((1, 1), jnp.float32),
            jax.ShapeDtypeStruct((1, NG), jnp.float32),
            jax.ShapeDtypeStruct((1, NG), jnp.float32),
        ],
        scratch_shapes=[pltpu.VMEM((NG, 8), jnp.float32)],
        compiler_params=pltpu.CompilerParams(
            dimension_semantics=("arbitrary",)),
    )(sm, cnt, ssum)

    disp_t, comb_t = pl.pallas_call(
        _p3_body,
        grid=(NB,),
        in_specs=[pl.BlockSpec((8, TB), lambda j: (0, j))],
        out_specs=[
            pl.BlockSpec((NG, CAP, TB), lambda j: (0, 0, j)),
            pl.BlockSpec((NG, CAP, TB), lambda j: (0, 0, j)),
        ],
        out_shape=[
            jax.ShapeDtypeStruct((NG, CAP, GS), jnp.float32),
            jax.ShapeDtypeStruct((NG, CAP, GS), jnp.float32),
        ],
        compiler_params=pltpu.CompilerParams(
            dimension_semantics=("parallel",)),
    )(r8)

    disp = jnp.transpose(disp_t, (2, 0, 1))[None]
    comb = jnp.transpose(comb_t, (2, 0, 1))[None]
    return (disp, comb, loss[0, 0], c1, c2)


# speedup vs baseline: 3.3918x; 1.2630x over previous
import jax
import jax.numpy as jnp
from jax.experimental import pallas as pl
from jax.experimental.pallas import tpu as pltpu

DIM_K = 2048
NG = 16
GS = 4096
CAP = 320
TB1 = 512
NB1 = GS // TB1
TB = 128
NB = GS // TB
EPS_ = 1e-9
NEG_BIG = -3.4e38


def _p1_body(x_ref, w_ref, sm_out, cnt_out, sum_out):
    raw = jnp.dot(x_ref[...], w_ref[...],
                  preferred_element_type=jnp.float32)
    m = jnp.max(raw, axis=1, keepdims=True)
    e = jnp.exp(raw - m)
    sm = e / jnp.sum(e, axis=1, keepdims=True)
    sm_out[...] = sm
    g1 = jnp.max(sm, axis=1, keepdims=True)
    iota = jax.lax.broadcasted_iota(jnp.int32, (TB1, NG), 1)
    i1 = jnp.min(jnp.where(sm == g1, iota, NG), axis=1, keepdims=True)
    mask1 = (iota == i1).astype(jnp.float32)
    cnt_out[...] = jnp.sum(mask1, axis=0, keepdims=True)[None]
    sum_out[...] = jnp.sum(sm, axis=0, keepdims=True)[None]


def _pf_body(sm_ref, cntp_ref, sump_ref, disp_ref, comb_ref,
             loss_ref, c1_ref, c2_ref, acc_ref):
    j = pl.program_id(0)

    @pl.when(j == 0)
    def _init():
        acc_ref[...] = jnp.zeros_like(acc_ref)

    cnt = jnp.sum(cntp_ref[...], axis=0)
    smt = jnp.transpose(sm_ref[...])
    eiota = jax.lax.broadcasted_iota(jnp.int32, (NG, TB), 0)
    g1 = jnp.max(smt, axis=0, keepdims=True)
    i1 = jnp.min(jnp.where(smt == g1, eiota, NG), axis=0, keepdims=True)
    masked = jnp.where(eiota == i1, NEG_BIG, smt)
    g2 = jnp.max(masked, axis=0, keepdims=True)
    i2 = jnp.min(jnp.where(masked == g2, eiota, NG), axis=0, keepdims=True)
    mask1 = (eiota == i1).astype(jnp.float32)
    mask2 = (eiota == i2).astype(jnp.float32)

    r = jax.lax.broadcasted_iota(jnp.int32, (TB, TB), 0)
    c = jax.lax.broadcasted_iota(jnp.int32, (TB, TB), 1)
    triu = (r < c).astype(jnp.float32)
    prev1 = jnp.dot(mask1, triu, preferred_element_type=jnp.float32)
    prev2 = jnp.dot(mask2, triu, preferred_element_type=jnp.float32)

    c1_run = acc_ref[:, 0:1]
    c2_run = acc_ref[:, 1:2]
    m1cnt = jnp.minimum(jnp.transpose(cnt), float(CAP))

    pos1 = jnp.sum((c1_run + prev1) * mask1, axis=0, keepdims=True)
    keep1 = (pos1 < float(CAP)).astype(jnp.float32)
    pos2 = jnp.sum((c2_run + prev2 + m1cnt) * mask2, axis=0, keepdims=True)
    keep2 = (pos2 < float(CAP)).astype(jnp.float32)

    acc_ref[:, 0:1] += jnp.sum(mask1, axis=1, keepdims=True)
    acc_ref[:, 1:2] += jnp.sum(mask2, axis=1, keepdims=True)
    acc_ref[:, 2:3] += jnp.sum(mask2 * keep2, axis=1, keepdims=True)

    denom = g1 + g2 + EPS_
    g1k = (g1 / denom) * keep1
    g2k = (g2 / denom) * keep2
    d1 = (g1k != 0.0).astype(jnp.float32)
    d2 = (g2k != 0.0).astype(jnp.float32)

    i1f = i1.astype(jnp.float32)
    i2f = i2.astype(jnp.float32)
    piota = jax.lax.broadcasted_iota(jnp.int32, (CAP, TB), 0).astype(
        jnp.float32)
    for e in range(NG):
        ef = float(e)
        is1 = i1f == ef
        is2 = i2f == ef
        pos_e = jnp.where(is1, pos1, jnp.where(is2, pos2, -1.0))
        val_e = jnp.where(is1, g1k, jnp.where(is2, g2k, 0.0))
        dva_e = jnp.where(is1, d1, jnp.where(is2, d2, 0.0))
        b = piota == pos_e
        comb_ref[e] = jnp.where(b, val_e, 0.0)
        disp_ref[e] = jnp.where(b, dva_e, 0.0)

    @pl.when(j == NB - 1)
    def _fin():
        c1_ref[...] = jnp.minimum(cnt, float(CAP))
        c2_ref[...] = jnp.transpose(acc_ref[:, 2:3])
        ssum = jnp.sum(sump_ref[...], axis=0)
        loss_ref[...] = jnp.sum(cnt * ssum, axis=1, keepdims=True) * (
            float(NG) / (float(GS) * float(GS)))


def kernel(x, w_gating):
    x2 = x.reshape(GS, DIM_K)
    sm, cntp, sump = pl.pallas_call(
        _p1_body,
        grid=(NB1,),
        in_specs=[
            pl.BlockSpec((TB1, DIM_K), lambda i: (i, 0)),
            pl.BlockSpec((DIM_K, NG), lambda i: (0, 0)),
        ],
        out_specs=[
            pl.BlockSpec((TB1, NG), lambda i: (i, 0)),
            pl.BlockSpec((1, 1, NG), lambda i: (i, 0, 0)),
            pl.BlockSpec((1, 1, NG), lambda i: (i, 0, 0)),
        ],
        out_shape=[
            jax.ShapeDtypeStruct((GS, NG), jnp.float32),
            jax.ShapeDtypeStruct((NB1, 1, NG), jnp.float32),
            jax.ShapeDtypeStruct((NB1, 1, NG), jnp.float32),
        ],
        compiler_params=pltpu.CompilerParams(
            dimension_semantics=("parallel",)),
    )(x2, w_gating)

    disp_t, comb_t, loss, c1, c2 = pl.pallas_call(
        _pf_body,
        grid=(NB,),
        in_specs=[
            pl.BlockSpec((TB, NG), lambda j: (j, 0)),
            pl.BlockSpec((NB1, 1, NG), lambda j: (0, 0, 0)),
            pl.BlockSpec((NB1, 1, NG), lambda j: (0, 0, 0)),
        ],
        out_specs=[
            pl.BlockSpec((NG, CAP, TB), lambda j: (0, 0, j)),
            pl.BlockSpec((NG, CAP, TB), lambda j: (0, 0, j)),
            pl.BlockSpec((1, 1), lambda j: (0, 0)),
            pl.BlockSpec((1, NG), lambda j: (0, 0)),
            pl.BlockSpec((1, NG), lambda j: (0, 0)),
        ],
        out_shape=[
            jax.ShapeDtypeStruct((NG, CAP, GS), jnp.float32),
            jax.ShapeDtypeStruct((NG, CAP, GS), jnp.float32),
            jax.ShapeDtypeStruct((1, 1), jnp.float32),
            jax.ShapeDtypeStruct((1, NG), jnp.float32),
            jax.ShapeDtypeStruct((1, NG), jnp.float32),
        ],
        scratch_shapes=[pltpu.VMEM((NG, 8), jnp.float32)],
        compiler_params=pltpu.CompilerParams(
            dimension_semantics=("arbitrary",)),
    )(sm, cntp, sump)

    disp = jnp.transpose(disp_t, (2, 0, 1))[None]
    comb = jnp.transpose(comb_t, (2, 0, 1))[None]
    return (disp, comb, loss[0, 0], c1, c2)


# speedup vs baseline: 3.6154x; 1.0659x over previous
import jax
import jax.numpy as jnp
from jax.experimental import pallas as pl
from jax.experimental.pallas import tpu as pltpu

DIM_K = 2048
NG = 16
GS = 4096
CAP = 320
TB1 = 1024
NB1 = GS // TB1
TB = 128
NB = GS // TB
EPS_ = 1e-9
NEG_BIG = -3.4e38


def _body(x_ref, w_ref, disp_ref, comb_ref, loss_ref, c1_ref, c2_ref,
          sm_s, acc_ref, acc2_ref):
    i = pl.program_id(0)

    @pl.when(i == 0)
    def _init():
        acc_ref[...] = jnp.zeros_like(acc_ref)
        acc2_ref[...] = jnp.zeros_like(acc2_ref)

    @pl.when(i < NB1)
    def _phase1():
        raw = jnp.dot(x_ref[...], w_ref[...],
                      preferred_element_type=jnp.float32)
        m = jnp.max(raw, axis=1, keepdims=True)
        e = jnp.exp(raw - m)
        sm = e / jnp.sum(e, axis=1, keepdims=True)
        sm_s[pl.ds(i * TB1, TB1), :] = sm
        g1 = jnp.max(sm, axis=1, keepdims=True)
        iota = jax.lax.broadcasted_iota(jnp.int32, (TB1, NG), 1)
        i1 = jnp.min(jnp.where(sm == g1, iota, NG), axis=1, keepdims=True)
        mask1 = (iota == i1).astype(jnp.float32)
        acc2_ref[0:1, :] += jnp.sum(mask1, axis=0, keepdims=True)
        acc2_ref[1:2, :] += jnp.sum(sm, axis=0, keepdims=True)

    @pl.when(i >= NB1)
    def _phase2():
        j = i - NB1
        cnt = acc2_ref[0:1, :]
        smt = jnp.transpose(sm_s[pl.ds(j * TB, TB), :])
        eiota = jax.lax.broadcasted_iota(jnp.int32, (NG, TB), 0)
        g1 = jnp.max(smt, axis=0, keepdims=True)
        i1 = jnp.min(jnp.where(smt == g1, eiota, NG), axis=0, keepdims=True)
        masked = jnp.where(eiota == i1, NEG_BIG, smt)
        g2 = jnp.max(masked, axis=0, keepdims=True)
        i2 = jnp.min(jnp.where(masked == g2, eiota, NG), axis=0,
                     keepdims=True)
        mask1 = (eiota == i1).astype(jnp.float32)
        mask2 = (eiota == i2).astype(jnp.float32)

        r = jax.lax.broadcasted_iota(jnp.int32, (TB, TB), 0)
        c = jax.lax.broadcasted_iota(jnp.int32, (TB, TB), 1)
        triu = (r < c).astype(jnp.float32)
        prev1 = jnp.dot(mask1, triu, preferred_element_type=jnp.float32)
        prev2 = jnp.dot(mask2, triu, preferred_element_type=jnp.float32)

        c1_run = acc_ref[:, 0:1]
        c2_run = acc_ref[:, 1:2]
        m1cnt = jnp.minimum(jnp.transpose(cnt), float(CAP))

        pos1 = jnp.sum((c1_run + prev1) * mask1, axis=0, keepdims=True)
        keep1 = (pos1 < float(CAP)).astype(jnp.float32)
        pos2 = jnp.sum((c2_run + prev2 + m1cnt) * mask2, axis=0,
                       keepdims=True)
        keep2 = (pos2 < float(CAP)).astype(jnp.float32)

        acc_ref[:, 0:1] += jnp.sum(mask1, axis=1, keepdims=True)
        acc_ref[:, 1:2] += jnp.sum(mask2, axis=1, keepdims=True)
        acc_ref[:, 2:3] += jnp.sum(mask2 * keep2, axis=1, keepdims=True)

        denom = g1 + g2 + EPS_
        g1k = (g1 / denom) * keep1
        g2k = (g2 / denom) * keep2
        d1 = (g1k != 0.0).astype(jnp.float32)
        d2 = (g2k != 0.0).astype(jnp.float32)

        i1f = i1.astype(jnp.float32)
        i2f = i2.astype(jnp.float32)
        piota = jax.lax.broadcasted_iota(jnp.int32, (CAP, TB), 0).astype(
            jnp.float32)
        for e in range(NG):
            ef = float(e)
            is1 = i1f == ef
            is2 = i2f == ef
            pos_e = jnp.where(is1, pos1, jnp.where(is2, pos2, -1.0))
            val_e = jnp.where(is1, g1k, jnp.where(is2, g2k, 0.0))
            dva_e = jnp.where(is1, d1, jnp.where(is2, d2, 0.0))
            b = piota == pos_e
            comb_ref[e] = jnp.where(b, val_e, 0.0)
            disp_ref[e] = jnp.where(b, dva_e, 0.0)

        @pl.when(j == NB - 1)
        def _fin():
            c1_ref[...] = jnp.minimum(cnt, float(CAP))
            c2_ref[...] = jnp.transpose(acc_ref[:, 2:3])
            loss_ref[...] = jnp.sum(cnt * acc2_ref[1:2, :], axis=1,
                                    keepdims=True) * (
                                        float(NG) / (float(GS) * float(GS)))


def kernel(x, w_gating):
    x2 = x.reshape(GS, DIM_K)
    disp_t, comb_t, loss, c1, c2 = pl.pallas_call(
        _body,
        grid=(NB1 + NB,),
        in_specs=[
            pl.BlockSpec((TB1, DIM_K),
                         lambda i: (jnp.minimum(i, NB1 - 1), 0)),
            pl.BlockSpec((DIM_K, NG), lambda i: (0, 0)),
        ],
        out_specs=[
            pl.BlockSpec((NG, CAP, TB),
                         lambda i: (0, 0, jnp.maximum(i - NB1, 0))),
            pl.BlockSpec((NG, CAP, TB),
                         lambda i: (0, 0, jnp.maximum(i - NB1, 0))),
            pl.BlockSpec((1, 1), lambda i: (0, 0)),
            pl.BlockSpec((1, NG), lambda i: (0, 0)),
            pl.BlockSpec((1, NG), lambda i: (0, 0)),
        ],
        out_shape=[
            jax.ShapeDtypeStruct((NG, CAP, GS), jnp.float32),
            jax.ShapeDtypeStruct((NG, CAP, GS), jnp.float32),
            jax.ShapeDtypeStruct((1, 1), jnp.float32),
            jax.ShapeDtypeStruct((1, NG), jnp.float32),
            jax.ShapeDtypeStruct((1, NG), jnp.float32),
        ],
        scratch_shapes=[
            pltpu.VMEM((GS, NG), jnp.float32),
            pltpu.VMEM((NG, 8), jnp.float32),
            pltpu.VMEM((2, NG), jnp.float32),
        ],
        compiler_params=pltpu.CompilerParams(
            dimension_semantics=("arbitrary",)),
    )(x2, w_gating)

    disp = jnp.transpose(disp_t, (2, 0, 1))[None]
    comb = jnp.transpose(comb_t, (2, 0, 1))[None]
    return (disp, comb, loss[0, 0], c1, c2)
